# baseline (speedup 1.0000x reference)
import jax
import jax.numpy as jnp
from jax import lax
from jax.experimental import pallas as pl
from jax.experimental.pallas import tpu as pltpu
from jax.experimental.pallas import tpu_sc as plsc

_B, _L = 16384, 20
_DLOC, _DTIME = 64, 32
_DOUT = _DLOC + _DTIME
_NC, _NS = 2, 16
_NW = _NC * _NS
_NB = 8
_CT = _NB * _L
_MBB = 128
_MBT = _MBB * _L
_CPM = _MBB // _NB
_B_PER_W = _B // _NW
_NMB = _B_PER_W // _MBB


def _body(x_hbm, t_hbm, loc0, loc1, loc2, tw0, tw1, out_hbm,
          xv, tv, idx_v, b0, b1, b2, tb0, tb1, out_v, sg0, sg1, ss0, ss1):
    wid = lax.axis_index("s") * _NC + lax.axis_index("c")
    sg = (sg0, sg1)
    ss = (ss0, ss1)

    def gather_descs(slot):
        s = sg[slot]
        ds = []
        for tbl, buf, k in ((loc0, b0, 0), (loc1, b1, 1), (loc2, b2, 2),
                            (tw0, tb0, 3), (tw1, tb1, 4)):
            ds.append(pltpu.make_async_copy(
                tbl.at[idx_v.at[slot, k, pl.ds(0, 128)]],
                buf.at[slot, pl.ds(0, 128)], s))
            ds.append(pltpu.make_async_copy(
                tbl.at[idx_v.at[slot, k, pl.ds(128, 32)]],
                buf.at[slot, pl.ds(128, 32)], s))
        return ds

    def extract(slot, c):
        for g in range(_CT // 16):
            tok = lax.iota(jnp.int32, 16) + (c * _CT + g * 16)
            x3 = tok * 3
            t2 = tok * 2
            for k in range(3):
                idx_v[slot, k, pl.ds(g * 16, 16)] = plsc.load_gather(
                    xv, [x3 + k])
            for k in range(2):
                idx_v[slot, 3 + k, pl.ds(g * 16, 16)] = plsc.load_gather(
                    tv, [t2 + k])

    def fire(slot, c):
        extract(slot, c)
        for d in gather_descs(slot):
            d.start()

    def wait_gathers(slot):
        for d in gather_descs(slot):
            d.wait()

    def scatter_desc(slot, brow):
        return pltpu.make_async_copy(
            out_v.at[slot], out_hbm.at[pl.ds(brow, _NB)], ss[slot])

    def compute(slot):
        def rows(bb, _):
            def cols(ll, _):
                tok = bb * _L + ll
                for j in range(_DLOC // 16):
                    s = pl.ds(j * 16, 16)
                    out_v[slot, bb, ll, s] = (b0[slot, tok, s]
                                              + b1[slot, tok, s]
                                              + b2[slot, tok, s])
                for j in range(_DTIME // 16):
                    s = pl.ds(j * 16, 16)
                    out_v[slot, bb, ll, pl.ds(_DLOC + j * 16, 16)] = (
                        tb0[slot, tok, s] + tb1[slot, tok, s])
                return ()
            lax.fori_loop(0, _L, cols, ())
            return ()
        lax.fori_loop(0, _NB, rows, ())

    @pl.loop(0, _NMB)
    def megablock(mb):
        mb_brow = wid * _B_PER_W + mb * _MBB
        mb_tok = mb_brow * _L
        pltpu.sync_copy(x_hbm.at[pl.ds(mb_tok * 3, _MBT * 3)], xv)
        pltpu.sync_copy(t_hbm.at[pl.ds(mb_tok * 2, _MBT * 2)], tv)
        fire(0, 0)

        @pl.loop(0, _CPM, step=2)
        def pair(k):
            for b in range(2):
                lc = k + b
                nxt = lc + 1

                @pl.when(nxt < _CPM)
                def _():
                    fire(1 - b, nxt)

                wait_gathers(b)
                compute(b)

                @pl.when(mb * _CPM + lc >= 2)
                def _():
                    scatter_desc(b, mb_brow).wait()

                scatter_desc(b, mb_brow + lc * _NB).start()

    scatter_desc(0, wid * _B_PER_W).wait()
    scatter_desc(1, wid * _B_PER_W).wait()


def kernel(x, t, loc_w0, loc_w1, loc_w2, time_w0, time_w1):
    xflat = x.reshape(-1).astype(jnp.int32)
    tflat = t.reshape(-1).astype(jnp.int32)
    mesh = plsc.VectorSubcoreMesh(core_axis_name="c", subcore_axis_name="s",
                                  num_cores=_NC, num_subcores=_NS)
    return pl.kernel(
        _body,
        out_type=jax.ShapeDtypeStruct((_B, _L, _DOUT), jnp.float32),
        mesh=mesh,
        scratch_types=[
            pltpu.VMEM((_MBT * 3,), jnp.int32),
            pltpu.VMEM((_MBT * 2,), jnp.int32),
            pltpu.VMEM((2, 5, _CT), jnp.int32),
            pltpu.VMEM((2, _CT, _DLOC), jnp.float32),
            pltpu.VMEM((2, _CT, _DLOC), jnp.float32),
            pltpu.VMEM((2, _CT, _DLOC), jnp.float32),
            pltpu.VMEM((2, _CT, _DTIME), jnp.float32),
            pltpu.VMEM((2, _CT, _DTIME), jnp.float32),
            pltpu.VMEM((2, _NB, _L, _DOUT), jnp.float32),
            pltpu.SemaphoreType.DMA,
            pltpu.SemaphoreType.DMA,
            pltpu.SemaphoreType.DMA,
            pltpu.SemaphoreType.DMA,
        ],
        compiler_params=pltpu.CompilerParams(use_tc_tiling_on_sc=False, needs_layout_passes=False),
    )(xflat, tflat, loc_w0, loc_w1, loc_w2, time_w0, time_w1)


# speedup vs baseline: 1.5264x; 1.5264x over previous
import jax
import jax.numpy as jnp
from jax import lax
from jax.experimental import pallas as pl
from jax.experimental.pallas import tpu as pltpu
from jax.experimental.pallas import tpu_sc as plsc

_B, _L = 16384, 20
_DLOC, _DTIME = 64, 32
_DOUT = _DLOC + _DTIME
_NC, _NS = 2, 16
_NW = _NC * _NS
_C = 128
_B_PER_W = _B // _NW
_NBC = _B_PER_W // _C


def _body(xT, tT, loc0, loc1, loc2, tw0, tw1, out_hbm,
          idx_v, b0, b1, b2, tb0, tb1, out_v, sg0, sg1, ss0, ss1):
    wid = lax.axis_index("s") * _NC + lax.axis_index("c")
    sg = (sg0, sg1)
    ss = (ss0, ss1)

    def gather_descs(slot, lc):
        s = sg[slot]
        return (
            pltpu.make_async_copy(loc0.at[idx_v.at[0, lc]], b0.at[slot], s),
            pltpu.make_async_copy(loc1.at[idx_v.at[1, lc]], b1.at[slot], s),
            pltpu.make_async_copy(loc2.at[idx_v.at[2, lc]], b2.at[slot], s),
            pltpu.make_async_copy(tw0.at[idx_v.at[3, lc]], tb0.at[slot], s),
            pltpu.make_async_copy(tw1.at[idx_v.at[4, lc]], tb1.at[slot], s),
        )

    def fire(slot, lc):
        for d in gather_descs(slot, lc):
            d.start()

    def wait_gathers(slot, lc):
        for d in gather_descs(slot, lc):
            d.wait()

    def scatter_desc(slot, lc, b0c):
        return pltpu.make_async_copy(
            out_v.at[slot], out_hbm.at[lc, pl.ds(b0c, _C)], ss[slot])

    def compute(slot):
        def row(i, _):
            for j in range(_DLOC // 16):
                s = pl.ds(j * 16, 16)
                out_v[slot, i, s] = (b0[slot, i, s] + b1[slot, i, s]
                                     + b2[slot, i, s])
            for j in range(_DTIME // 16):
                s = pl.ds(j * 16, 16)
                out_v[slot, i, pl.ds(_DLOC + j * 16, 16)] = (
                    tb0[slot, i, s] + tb1[slot, i, s])
            return ()
        lax.fori_loop(0, _C, row, ())

    @pl.loop(0, _NBC)
    def bchunk(bc):
        b0c = wid * _B_PER_W + bc * _C
        for k in range(3):
            pltpu.sync_copy(xT.at[k, :, pl.ds(b0c, _C)], idx_v.at[k])
        for k in range(2):
            pltpu.sync_copy(tT.at[:, k, pl.ds(b0c, _C)], idx_v.at[3 + k])
        fire(0, 0)

        @pl.loop(0, _L, step=2)
        def pair(k):
            for b in range(2):
                lc = k + b
                nxt = lc + 1

                @pl.when(nxt < _L)
                def _():
                    fire(1 - b, nxt)

                wait_gathers(b, lc)
                compute(b)

                @pl.when(bc * _L + lc >= 2)
                def _():
                    scatter_desc(b, lc, b0c).wait()

                scatter_desc(b, lc, b0c).start()

    scatter_desc(0, 0, wid * _B_PER_W).wait()
    scatter_desc(1, 0, wid * _B_PER_W).wait()


def kernel(x, t, loc_w0, loc_w1, loc_w2, time_w0, time_w1):
    xT = jnp.transpose(x, (2, 1, 0)).astype(jnp.int32)
    tT = jnp.transpose(t, (1, 2, 0)).astype(jnp.int32)
    mesh = plsc.VectorSubcoreMesh(core_axis_name="c", subcore_axis_name="s",
                                  num_cores=_NC, num_subcores=_NS)
    out2 = pl.kernel(
        _body,
        out_type=jax.ShapeDtypeStruct((_L, _B, _DOUT), jnp.float32),
        mesh=mesh,
        scratch_types=[
            pltpu.VMEM((5, _L, _C), jnp.int32),
            pltpu.VMEM((2, _C, _DLOC), jnp.float32),
            pltpu.VMEM((2, _C, _DLOC), jnp.float32),
            pltpu.VMEM((2, _C, _DLOC), jnp.float32),
            pltpu.VMEM((2, _C, _DTIME), jnp.float32),
            pltpu.VMEM((2, _C, _DTIME), jnp.float32),
            pltpu.VMEM((2, _C, _DOUT), jnp.float32),
            pltpu.SemaphoreType.DMA,
            pltpu.SemaphoreType.DMA,
            pltpu.SemaphoreType.DMA,
            pltpu.SemaphoreType.DMA,
        ],
        compiler_params=pltpu.CompilerParams(use_tc_tiling_on_sc=False),
    )(xT, tT, loc_w0, loc_w1, loc_w2, time_w0, time_w1)
    return jnp.transpose(out2, (1, 0, 2))
